# baseline (speedup 1.0000x reference)
import functools

import jax
import jax.numpy as jnp
from jax import lax
from jax.experimental import pallas as pl
from jax.experimental.pallas import tpu as pltpu
from jax.experimental.pallas import tpu_sc as plsc

_TOPK = 16
_NC, _NS = 2, 16
_NTILES = _NC * _NS
_NSUB = 4
_NR = _NTILES * _NSUB
_F = 128
_GC = 128
_BLK = 1024
_RING = 4
_FLUSH = 2048
_CHB = 1024


def _block_rows(n):
    for br in (400, 256, 250, 200, 128, 125, 100, 64, 50, 32, 25, 16, 8, 5, 4, 2, 1):
        if n % br == 0:
            return br
    return 1


def _scan_chunk(e):
    for c in (2000, 1600, 1280, 1024, 1000, 800, 640, 512, 400, 320, 256, 160, 128, 80, 64, 32, 16):
        if e % c == 0:
            return c
    return 16


def _weight_xpad_kernel(x_ref, w1_ref, b1_ref, w2_ref, b2_ref, w_ref, xpad_ref):
    x = x_ref[...]
    br, c = x.shape
    m = jnp.max(x, axis=1, keepdims=True)
    z = jnp.sum(jnp.exp(x - m), axis=1, keepdims=True)
    cols = jax.lax.broadcasted_iota(jnp.int32, (br, c), 1)
    work = x
    tvals = []
    for _ in range(_TOPK):
        mk = jnp.max(work, axis=1, keepdims=True)
        fi = jnp.min(jnp.where(work == mk, cols, c + 1), axis=1, keepdims=True)
        tvals.append(mk)
        work = jnp.where(cols == fi, -jnp.inf, work)
    t = jnp.concatenate(tvals, axis=1)
    t = jnp.exp(t - m) / z
    h = jnp.dot(t, w1_ref[...], preferred_element_type=jnp.float32) + b1_ref[...]
    h = jnp.where(h >= 0, h, 0.1 * h)
    zz = jnp.dot(h, w2_ref[...], preferred_element_type=jnp.float32) + b2_ref[...]
    mm = jnp.max(zz, axis=1, keepdims=True)
    e = jnp.exp(zz - mm)
    w_ref[...] = e / jnp.sum(e, axis=1, keepdims=True)
    xpad_ref[...] = jnp.concatenate(
        [x, jnp.zeros((br, _F - c), jnp.float32)], axis=1)


def _combine_kernel(x_ref, p1, p2, p3, p4, p5, w_ref, out_ref):
    x = x_ref[...]
    w = w_ref[...]
    c = x.shape[1]
    acc = w[:, 0:1] * x
    for k, p in enumerate((p1, p2, p3, p4, p5)):
        acc = acc + w[:, k + 1:k + 2] * p[:, :c]
    m = jnp.max(acc, axis=1, keepdims=True)
    s = acc - m
    out_ref[...] = s - jnp.log(jnp.sum(jnp.exp(s), axis=1, keepdims=True))


def _bin_body(dst_hbm, src_hbm, val_hbm, bpk, bval, counts,
              dbuf, sbuf, vbuf, sga_p, sga_v, rbp, rbv,
              sgb_p, sgb_v, cbuf, *, e_total, ch, rna, rnb):
    w = lax.axis_index("s") * _NC + lax.axis_index("c")
    lo = w * rna
    hi = lo + rna
    nch = e_total // ch
    arow = _NR + w

    def chunk_body(i, carry):
        cur, gcur = carry
        base = pl.multiple_of(i * ch, 8)
        pltpu.sync_copy(dst_hbm.at[pl.ds(base, ch)], dbuf)
        pltpu.sync_copy(src_hbm.at[pl.ds(base, ch)], sbuf)
        pltpu.sync_copy(val_hbm.at[pl.ds(base, ch)], vbuf)

        def vbody(j, cur):
            d = dbuf[pl.ds(j * 16, 16)]
            m = (d >= lo) & (d < hi)
            pk = sbuf[pl.ds(j * 16, 16)] | ((d - lo) << 16)
            plsc.store_compressed(sga_p.at[pl.ds(cur, 16)], pk, mask=m)
            plsc.store_compressed(sga_v.at[pl.ds(cur, 16)],
                                  vbuf[pl.ds(j * 16, 16)], mask=m)
            return cur + plsc.all_reduce_population_count(m)[0]

        cur = lax.fori_loop(0, ch // 16, vbody, cur)

        def do_flush(args):
            cur, gcur = args
            gcur = pl.multiple_of(gcur, 8)
            pltpu.sync_copy(sga_p.at[pl.ds(0, _FLUSH)],
                            bpk.at[arow, pl.ds(gcur, _FLUSH)])
            pltpu.sync_copy(sga_v.at[pl.ds(0, _FLUSH)],
                            bval.at[arow, pl.ds(gcur, _FLUSH)])

            def mv(j, _):
                sga_p[pl.ds(j * 16, 16)] = sga_p[pl.ds(_FLUSH + j * 16, 16)]
                sga_v[pl.ds(j * 16, 16)] = sga_v[pl.ds(_FLUSH + j * 16, 16)]
                return 0

            lax.fori_loop(0, ch // 16, mv, 0)
            return cur - _FLUSH, gcur + _FLUSH

        return lax.cond(cur >= _FLUSH, do_flush, lambda a: a, (cur, gcur))

    cur, gcur = lax.fori_loop(
        0, nch, chunk_body, (jnp.int32(0), jnp.int32(0)))

    sent = jnp.full((16,), 0x7FF << 16, jnp.int32)
    zf = jnp.zeros((16,), jnp.float32)
    for j in range(_CHB // 16):
        sga_p[pl.ds(cur + j * 16, 16)] = sent
        sga_v[pl.ds(cur + j * 16, 16)] = zf
    cur = ((cur + _CHB - 1) // _CHB) * _CHB

    def fflush(args):
        cur, gcur = args
        gcur = pl.multiple_of(gcur, 8)
        pltpu.sync_copy(sga_p.at[pl.ds(0, _FLUSH)],
                        bpk.at[arow, pl.ds(gcur, _FLUSH)])
        pltpu.sync_copy(sga_v.at[pl.ds(0, _FLUSH)],
                        bval.at[arow, pl.ds(gcur, _FLUSH)])
        return args

    lax.cond(cur > 0, fflush, lambda a: a, (cur, gcur))
    cnta = gcur + cur

    def sub_chunk(i, carry):
        base = pl.multiple_of(i * _CHB, 8)
        pltpu.sync_copy(bpk.at[arow, pl.ds(base, _CHB)], rbp)
        pltpu.sync_copy(bval.at[arow, pl.ds(base, _CHB)], rbv)
        curs = list(carry[:_NSUB])
        gcurs = list(carry[_NSUB:])

        def vbody(j, curs):
            curs = list(curs)
            pk = rbp[pl.ds(j * 16, 16)]
            vv = rbv[pl.ds(j * 16, 16)]
            dla = pk >> 16
            for k in range(_NSUB):
                m = (dla >= k * rnb) & (dla < (k + 1) * rnb)
                adj = pk - (k * rnb) * 65536
                plsc.store_compressed(sgb_p.at[k, pl.ds(curs[k], 16)],
                                      adj, mask=m)
                plsc.store_compressed(sgb_v.at[k, pl.ds(curs[k], 16)],
                                      vv, mask=m)
                curs[k] = curs[k] + plsc.all_reduce_population_count(m)[0]
            return tuple(curs)

        curs = lax.fori_loop(0, _CHB // 16, vbody, tuple(curs))
        news = []
        for k in range(_NSUB):
            def do_flush(args, k=k):
                cur, gcur = args
                gcur = pl.multiple_of(gcur, 8)
                pltpu.sync_copy(sgb_p.at[k, pl.ds(0, _FLUSH)],
                                bpk.at[w * _NSUB + k, pl.ds(gcur, _FLUSH)])
                pltpu.sync_copy(sgb_v.at[k, pl.ds(0, _FLUSH)],
                                bval.at[w * _NSUB + k, pl.ds(gcur, _FLUSH)])

                def mv(j, _, k=k):
                    sgb_p[k, pl.ds(j * 16, 16)] = \
                        sgb_p[k, pl.ds(_FLUSH + j * 16, 16)]
                    sgb_v[k, pl.ds(j * 16, 16)] = \
                        sgb_v[k, pl.ds(_FLUSH + j * 16, 16)]
                    return 0

                lax.fori_loop(0, _CHB // 16, mv, 0)
                return cur - _FLUSH, gcur + _FLUSH

            news.append(lax.cond(curs[k] >= _FLUSH, do_flush, lambda a: a,
                                 (curs[k], gcurs[k])))
        return tuple(n[0] for n in news) + tuple(n[1] for n in news)

    zero8 = tuple(jnp.int32(0) for _ in range(2 * _NSUB))
    fin = lax.fori_loop(0, cnta // _CHB, sub_chunk, zero8)

    dum = jnp.full((16,), rnb << 16, jnp.int32)
    zf16 = jnp.zeros((16,), jnp.float32)
    for k in range(_NSUB):
        cur, gcur = fin[k], fin[_NSUB + k]
        for j in range(_BLK // 16):
            sgb_p[k, pl.ds(cur + j * 16, 16)] = dum
            sgb_v[k, pl.ds(cur + j * 16, 16)] = zf16
        cur = ((cur + _BLK - 1) // _BLK) * _BLK

        def fflushb(args, k=k):
            cur, gcur = args
            gcur = pl.multiple_of(gcur, 8)
            pltpu.sync_copy(sgb_p.at[k, pl.ds(0, _FLUSH)],
                            bpk.at[w * _NSUB + k, pl.ds(gcur, _FLUSH)])
            pltpu.sync_copy(sgb_v.at[k, pl.ds(0, _FLUSH)],
                            bval.at[w * _NSUB + k, pl.ds(gcur, _FLUSH)])
            return args

        lax.cond(cur > 0, fflushb, lambda a: a, (cur, gcur))
        cbuf[k, pl.ds(0, 16)] = jnp.full((16,), gcur + cur, jnp.int32)
    pltpu.sync_copy(cbuf.at[pl.ds(0, _NSUB), pl.ds(0, 8)],
                    counts.at[pl.ds(w * _NSUB, _NSUB)])


def _hop_body(pre, bpk, bval, counts, out,
              pkb, vb, srcb, rows0, rows1, rows2, rows3, acc, cnt16,
              sem0, sem1, sem2, sem3, *, rnb, racc):
    w = lax.axis_index("s") * _NC + lax.axis_index("c")
    rows = (rows0, rows1, rows2, rows3)
    sems = (sem0, sem1, sem2, sem3)
    iota = lax.iota(jnp.int32, 16)
    zf = jnp.zeros((16,), jnp.float32)
    nacc = racc * _F // 16

    def range_body(r, _):
        row = w * _NSUB + r
        pltpu.sync_copy(counts.at[row], cnt16.at[pl.ds(0, 8)])
        cnt = cnt16[pl.ds(0, 16)][0]
        nblk = cnt // _BLK

        def zero_body(i, _):
            for j in range(8):
                acc[pl.ds((i * 8 + j) * 16, 16)] = zf
            return 0

        lax.fori_loop(0, nacc // 8, zero_body, 0)

        def blk_body(i, _):
            base = pl.multiple_of(i * _BLK, 8)
            pltpu.sync_copy(bpk.at[row, pl.ds(base, _BLK)], pkb)
            pltpu.sync_copy(bval.at[row, pl.ds(base, _BLK)], vb)

            def unpack(j, _):
                for j2 in range(4):
                    jj = (j * 4 + j2) * 16
                    srcb[pl.ds(jj, 16)] = pkb[pl.ds(jj, 16)] & 0xFFFF
                return 0

            lax.fori_loop(0, _BLK // 64, unpack, 0)

            for c in range(_BLK // _GC):
                b = c % _RING
                pltpu.async_copy(
                    pre.at[srcb.at[pl.ds(c * _GC, _GC)]], rows[b],
                    sems[b]).wait()

                def fma_body(g, _, c=c, b=b):
                    off = c * _GC + g * 16
                    pk = pkb[pl.ds(off, 16)]
                    vv = vb[pl.ds(off, 16)]
                    dlv = (pk >> 16) << 7
                    for u in range(16):
                        base = dlv[u]
                        vsc = vv[u]
                        e = g * 16 + u
                        rvs = [rows[b][e, pl.ds(ps * 16, 16)]
                               for ps in range(8)]
                        msgs = [vsc * rv for rv in rvs]
                        for ps in range(8):
                            plsc.addupdate(
                                acc.at[pl.ds(base + ps * 16, 16)], msgs[ps])
                    return 0

                lax.fori_loop(0, _GC // 16, fma_body, 0)
            return 0

        lax.fori_loop(0, nblk, blk_body, 0)
        rowbase = pl.multiple_of(row * (rnb * _F), 8)
        pltpu.sync_copy(acc.at[pl.ds(0, rnb * _F)],
                        out.at[pl.ds(rowbase, rnb * _F)])
        return 0

    lax.fori_loop(0, _NSUB, range_body, 0)


def kernel(x, adj_vals, W1, b1, W2, b2, edge_index):
    n, c = x.shape
    e_total = adj_vals.shape[0]
    degree = W2.shape[1]
    br = _block_rows(n)
    grid = n // br
    rnb = -(-n // _NR)
    rna = _NSUB * rnb
    npad = _NR * rnb
    racc = ((rnb + 1 + 7) // 8) * 8
    eb = e_total + 2 * _FLUSH + _BLK
    ch = _scan_chunk(e_total)

    weight, xpad = pl.pallas_call(
        _weight_xpad_kernel,
        grid=(grid,),
        in_specs=[
            pl.BlockSpec((br, c), lambda i: (i, 0)),
            pl.BlockSpec(W1.shape, lambda i: (0, 0)),
            pl.BlockSpec((1, W1.shape[1]), lambda i: (0, 0)),
            pl.BlockSpec(W2.shape, lambda i: (0, 0)),
            pl.BlockSpec((1, W2.shape[1]), lambda i: (0, 0)),
        ],
        out_specs=[
            pl.BlockSpec((br, degree), lambda i: (i, 0)),
            pl.BlockSpec((br, _F), lambda i: (i, 0)),
        ],
        out_shape=[
            jax.ShapeDtypeStruct((n, degree), jnp.float32),
            jax.ShapeDtypeStruct((npad, _F), jnp.float32),
        ],
    )(x, W1, b1.reshape(1, -1), W2, b2.reshape(1, -1))

    mesh = plsc.VectorSubcoreMesh(core_axis_name="c", subcore_axis_name="s",
                                  num_cores=_NC, num_subcores=_NS)
    sc_params = pltpu.CompilerParams(use_tc_tiling_on_sc=False,
                                     needs_layout_passes=False)
    i32 = jnp.int32
    stga = _FLUSH + ch + _CHB + 16
    stgb = _FLUSH + _CHB + _BLK + 16
    bin_fn = pl.kernel(
        functools.partial(_bin_body, e_total=e_total, ch=ch, rna=rna, rnb=rnb),
        out_type=[
            jax.ShapeDtypeStruct((_NR + _NTILES, eb), i32),
            jax.ShapeDtypeStruct((_NR + _NTILES, eb), jnp.float32),
            jax.ShapeDtypeStruct((_NR, 8), i32),
        ],
        mesh=mesh,
        scratch_types=[
            pltpu.VMEM((ch,), i32),
            pltpu.VMEM((ch,), i32),
            pltpu.VMEM((ch,), jnp.float32),
            pltpu.VMEM((stga,), i32),
            pltpu.VMEM((stga,), jnp.float32),
            pltpu.VMEM((_CHB,), i32),
            pltpu.VMEM((_CHB,), jnp.float32),
            pltpu.VMEM((_NSUB, stgb), i32),
            pltpu.VMEM((_NSUB, stgb), jnp.float32),
            pltpu.VMEM((_NSUB, 16), i32),
        ],
        compiler_params=sc_params,
    )
    bpk, bval, counts = bin_fn(edge_index[0], edge_index[1], adj_vals)

    hop_fn = pl.kernel(
        functools.partial(_hop_body, rnb=rnb, racc=racc),
        out_type=jax.ShapeDtypeStruct((npad * _F,), jnp.float32),
        mesh=mesh,
        scratch_types=[
            pltpu.VMEM((_BLK,), i32),
            pltpu.VMEM((_BLK,), jnp.float32),
            pltpu.VMEM((_BLK,), i32),
            pltpu.VMEM((_GC, _F), jnp.float32),
            pltpu.VMEM((_GC, _F), jnp.float32),
            pltpu.VMEM((_GC, _F), jnp.float32),
            pltpu.VMEM((_GC, _F), jnp.float32),
            pltpu.VMEM((racc * _F,), jnp.float32),
            pltpu.VMEM((16,), i32),
            pltpu.SemaphoreType.DMA,
            pltpu.SemaphoreType.DMA,
            pltpu.SemaphoreType.DMA,
            pltpu.SemaphoreType.DMA,
        ],
        compiler_params=sc_params,
    )

    pres = []
    pre = xpad
    for _ in range(degree - 1):
        pre = hop_fn(pre, bpk, bval, counts).reshape(npad, _F)
        pres.append(pre)

    out = pl.pallas_call(
        _combine_kernel,
        grid=(grid,),
        in_specs=[pl.BlockSpec((br, c), lambda i: (i, 0))]
        + [pl.BlockSpec((br, _F), lambda i: (i, 0))] * (degree - 1)
        + [pl.BlockSpec((br, degree), lambda i: (i, 0))],
        out_specs=pl.BlockSpec((br, c), lambda i: (i, 0)),
        out_shape=jax.ShapeDtypeStruct((n, c), jnp.float32),
    )(x, *pres, weight)
    return out


# speedup vs baseline: 1.0001x; 1.0001x over previous
import functools

import jax
import jax.numpy as jnp
from jax import lax
from jax.experimental import pallas as pl
from jax.experimental.pallas import tpu as pltpu
from jax.experimental.pallas import tpu_sc as plsc

_TOPK = 16
_NC, _NS = 2, 16
_NTILES = _NC * _NS
_NSUB = 4
_NR = _NTILES * _NSUB
_F = 128
_GC = 128
_BLK = 1024
_RING = 4
_FLUSH = 2048
_CHB = 1024


def _block_rows(n):
    for br in (400, 256, 250, 200, 128, 125, 100, 64, 50, 32, 25, 16, 8, 5, 4, 2, 1):
        if n % br == 0:
            return br
    return 1


def _scan_chunk(e):
    for c in (2000, 1600, 1280, 1024, 1000, 800, 640, 512, 400, 320, 256, 160, 128, 80, 64, 32, 16):
        if e % c == 0:
            return c
    return 16


def _weight_xpad_kernel(x_ref, w1_ref, b1_ref, w2_ref, b2_ref, w_ref, xpad_ref):
    x = x_ref[...]
    br, c = x.shape
    m = jnp.max(x, axis=1, keepdims=True)
    z = jnp.sum(jnp.exp(x - m), axis=1, keepdims=True)
    cols = jax.lax.broadcasted_iota(jnp.int32, (br, c), 1)
    work = x
    tvals = []
    for _ in range(_TOPK):
        mk = jnp.max(work, axis=1, keepdims=True)
        fi = jnp.min(jnp.where(work == mk, cols, c + 1), axis=1, keepdims=True)
        tvals.append(mk)
        work = jnp.where(cols == fi, -jnp.inf, work)
    t = jnp.concatenate(tvals, axis=1)
    t = jnp.exp(t - m) / z
    h = jnp.dot(t, w1_ref[...], preferred_element_type=jnp.float32) + b1_ref[...]
    h = jnp.where(h >= 0, h, 0.1 * h)
    zz = jnp.dot(h, w2_ref[...], preferred_element_type=jnp.float32) + b2_ref[...]
    mm = jnp.max(zz, axis=1, keepdims=True)
    e = jnp.exp(zz - mm)
    w_ref[...] = e / jnp.sum(e, axis=1, keepdims=True)
    xpad_ref[...] = jnp.concatenate(
        [x, jnp.zeros((br, _F - c), jnp.float32)], axis=1)


def _combine_kernel(x_ref, p1, p2, p3, p4, p5, w_ref, out_ref):
    x = x_ref[...]
    w = w_ref[...]
    c = x.shape[1]
    acc = w[:, 0:1] * x
    for k, p in enumerate((p1, p2, p3, p4, p5)):
        acc = acc + w[:, k + 1:k + 2] * p[:, :c]
    m = jnp.max(acc, axis=1, keepdims=True)
    s = acc - m
    out_ref[...] = s - jnp.log(jnp.sum(jnp.exp(s), axis=1, keepdims=True))


def _bin_body(dst_hbm, src_hbm, val_hbm, bpk, bval, counts,
              dbuf, sbuf, vbuf, sga_p, sga_v, rbp, rbv,
              sgb_p, sgb_v, cbuf, *, e_total, ch, rna, rnb):
    w = lax.axis_index("s") * _NC + lax.axis_index("c")
    lo = w * rna
    hi = lo + rna
    nch = e_total // ch
    arow = _NR + w

    def chunk_body(i, carry):
        cur, gcur = carry
        base = pl.multiple_of(i * ch, 8)
        pltpu.sync_copy(dst_hbm.at[pl.ds(base, ch)], dbuf)
        pltpu.sync_copy(src_hbm.at[pl.ds(base, ch)], sbuf)
        pltpu.sync_copy(val_hbm.at[pl.ds(base, ch)], vbuf)

        def vbody(j, cur):
            d = dbuf[pl.ds(j * 16, 16)]
            m = (d >= lo) & (d < hi)
            pk = sbuf[pl.ds(j * 16, 16)] | ((d - lo) << 16)
            plsc.store_compressed(sga_p.at[pl.ds(cur, 16)], pk, mask=m)
            plsc.store_compressed(sga_v.at[pl.ds(cur, 16)],
                                  vbuf[pl.ds(j * 16, 16)], mask=m)
            return cur + plsc.all_reduce_population_count(m)[0]

        cur = lax.fori_loop(0, ch // 16, vbody, cur)

        def do_flush(args):
            cur, gcur = args
            gcur = pl.multiple_of(gcur, 8)
            pltpu.sync_copy(sga_p.at[pl.ds(0, _FLUSH)],
                            bpk.at[arow, pl.ds(gcur, _FLUSH)])
            pltpu.sync_copy(sga_v.at[pl.ds(0, _FLUSH)],
                            bval.at[arow, pl.ds(gcur, _FLUSH)])

            def mv(j, _):
                sga_p[pl.ds(j * 16, 16)] = sga_p[pl.ds(_FLUSH + j * 16, 16)]
                sga_v[pl.ds(j * 16, 16)] = sga_v[pl.ds(_FLUSH + j * 16, 16)]
                return 0

            lax.fori_loop(0, ch // 16, mv, 0)
            return cur - _FLUSH, gcur + _FLUSH

        return lax.cond(cur >= _FLUSH, do_flush, lambda a: a, (cur, gcur))

    cur, gcur = lax.fori_loop(
        0, nch, chunk_body, (jnp.int32(0), jnp.int32(0)))

    sent = jnp.full((16,), 0x7FF << 16, jnp.int32)
    zf = jnp.zeros((16,), jnp.float32)
    for j in range(_CHB // 16):
        sga_p[pl.ds(cur + j * 16, 16)] = sent
        sga_v[pl.ds(cur + j * 16, 16)] = zf
    cur = ((cur + _CHB - 1) // _CHB) * _CHB

    def fflush(args):
        cur, gcur = args
        gcur = pl.multiple_of(gcur, 8)
        pltpu.sync_copy(sga_p.at[pl.ds(0, _FLUSH)],
                        bpk.at[arow, pl.ds(gcur, _FLUSH)])
        pltpu.sync_copy(sga_v.at[pl.ds(0, _FLUSH)],
                        bval.at[arow, pl.ds(gcur, _FLUSH)])
        return args

    lax.cond(cur > 0, fflush, lambda a: a, (cur, gcur))
    cnta = gcur + cur

    def sub_chunk(i, carry):
        base = pl.multiple_of(i * _CHB, 8)
        pltpu.sync_copy(bpk.at[arow, pl.ds(base, _CHB)], rbp)
        pltpu.sync_copy(bval.at[arow, pl.ds(base, _CHB)], rbv)
        curs = list(carry[:_NSUB])
        gcurs = list(carry[_NSUB:])

        def vbody(j, curs):
            curs = list(curs)
            pk = rbp[pl.ds(j * 16, 16)]
            vv = rbv[pl.ds(j * 16, 16)]
            dla = pk >> 16
            for k in range(_NSUB):
                m = (dla >= k * rnb) & (dla < (k + 1) * rnb)
                adj = pk - (k * rnb) * 65536
                plsc.store_compressed(sgb_p.at[k, pl.ds(curs[k], 16)],
                                      adj, mask=m)
                plsc.store_compressed(sgb_v.at[k, pl.ds(curs[k], 16)],
                                      vv, mask=m)
                curs[k] = curs[k] + plsc.all_reduce_population_count(m)[0]
            return tuple(curs)

        curs = lax.fori_loop(0, _CHB // 16, vbody, tuple(curs))
        news = []
        for k in range(_NSUB):
            def do_flush(args, k=k):
                cur, gcur = args
                gcur = pl.multiple_of(gcur, 8)
                pltpu.sync_copy(sgb_p.at[k, pl.ds(0, _FLUSH)],
                                bpk.at[w * _NSUB + k, pl.ds(gcur, _FLUSH)])
                pltpu.sync_copy(sgb_v.at[k, pl.ds(0, _FLUSH)],
                                bval.at[w * _NSUB + k, pl.ds(gcur, _FLUSH)])

                def mv(j, _, k=k):
                    sgb_p[k, pl.ds(j * 16, 16)] = \
                        sgb_p[k, pl.ds(_FLUSH + j * 16, 16)]
                    sgb_v[k, pl.ds(j * 16, 16)] = \
                        sgb_v[k, pl.ds(_FLUSH + j * 16, 16)]
                    return 0

                lax.fori_loop(0, _CHB // 16, mv, 0)
                return cur - _FLUSH, gcur + _FLUSH

            news.append(lax.cond(curs[k] >= _FLUSH, do_flush, lambda a: a,
                                 (curs[k], gcurs[k])))
        return tuple(n[0] for n in news) + tuple(n[1] for n in news)

    zero8 = tuple(jnp.int32(0) for _ in range(2 * _NSUB))
    fin = lax.fori_loop(0, cnta // _CHB, sub_chunk, zero8)

    dum = jnp.full((16,), rnb << 16, jnp.int32)
    zf16 = jnp.zeros((16,), jnp.float32)
    for k in range(_NSUB):
        cur, gcur = fin[k], fin[_NSUB + k]
        for j in range(_BLK // 16):
            sgb_p[k, pl.ds(cur + j * 16, 16)] = dum
            sgb_v[k, pl.ds(cur + j * 16, 16)] = zf16
        cur = ((cur + _BLK - 1) // _BLK) * _BLK

        def fflushb(args, k=k):
            cur, gcur = args
            gcur = pl.multiple_of(gcur, 8)
            pltpu.sync_copy(sgb_p.at[k, pl.ds(0, _FLUSH)],
                            bpk.at[w * _NSUB + k, pl.ds(gcur, _FLUSH)])
            pltpu.sync_copy(sgb_v.at[k, pl.ds(0, _FLUSH)],
                            bval.at[w * _NSUB + k, pl.ds(gcur, _FLUSH)])
            return args

        lax.cond(cur > 0, fflushb, lambda a: a, (cur, gcur))
        cbuf[k, pl.ds(0, 16)] = jnp.full((16,), gcur + cur, jnp.int32)
    pltpu.sync_copy(cbuf.at[pl.ds(0, _NSUB), pl.ds(0, 8)],
                    counts.at[pl.ds(w * _NSUB, _NSUB)])


def _hop_body(pre, bpk, bval, counts, out,
              pkb, vb, srcb, rows0, rows1, rows2, rows3, acc, cnt16,
              sem0, sem1, sem2, sem3, *, rnb, racc):
    w = lax.axis_index("s") * _NC + lax.axis_index("c")
    rows = (rows0, rows1, rows2, rows3)
    sems = (sem0, sem1, sem2, sem3)
    iota = lax.iota(jnp.int32, 16)
    zf = jnp.zeros((16,), jnp.float32)
    nacc = racc * _F // 16

    def range_body(r, _):
        row = w * _NSUB + r
        pltpu.sync_copy(counts.at[row], cnt16.at[pl.ds(0, 8)])
        cnt = cnt16[pl.ds(0, 16)][0]
        nblk = cnt // _BLK

        def zero_body(i, _):
            for j in range(8):
                acc[pl.ds((i * 8 + j) * 16, 16)] = zf
            return 0

        lax.fori_loop(0, nacc // 8, zero_body, 0)

        def blk_body(i, _):
            base = pl.multiple_of(i * _BLK, 8)
            pltpu.sync_copy(bpk.at[row, pl.ds(base, _BLK)], pkb)
            pltpu.sync_copy(bval.at[row, pl.ds(base, _BLK)], vb)

            for c in range(_BLK // _GC):
                for j in range(_GC // 16):
                    srcb[c, pl.ds(j * 16, 16)] = \
                        pkb[pl.ds(c * _GC + j * 16, 16)] & 0xFFFF

            for c in range(_BLK // _GC):
                b = c % _RING
                pltpu.async_copy(
                    pre.at[srcb.at[c]], rows[b], sems[b]).wait()

                def fma_body(g, _, c=c, b=b):
                    off = c * _GC + g * 16
                    pk = pkb[pl.ds(off, 16)]
                    vv = vb[pl.ds(off, 16)]
                    dlv = (pk >> 16) << 7
                    for u in range(16):
                        base = dlv[u]
                        vsc = vv[u]
                        e = g * 16 + u
                        rvs = [rows[b][e, pl.ds(ps * 16, 16)]
                               for ps in range(8)]
                        msgs = [vsc * rv for rv in rvs]
                        for ps in range(8):
                            plsc.addupdate(
                                acc.at[pl.ds(base + ps * 16, 16)], msgs[ps])
                    return 0

                lax.fori_loop(0, _GC // 16, fma_body, 0)
            return 0

        lax.fori_loop(0, nblk, blk_body, 0)
        rowbase = pl.multiple_of(row * (rnb * _F), 8)
        pltpu.sync_copy(acc.at[pl.ds(0, rnb * _F)],
                        out.at[pl.ds(rowbase, rnb * _F)])
        return 0

    lax.fori_loop(0, _NSUB, range_body, 0)


def kernel(x, adj_vals, W1, b1, W2, b2, edge_index):
    n, c = x.shape
    e_total = adj_vals.shape[0]
    degree = W2.shape[1]
    br = _block_rows(n)
    grid = n // br
    rnb = -(-n // _NR)
    rna = _NSUB * rnb
    npad = _NR * rnb
    racc = ((rnb + 1 + 7) // 8) * 8
    eb = e_total + 2 * _FLUSH + _BLK
    ch = _scan_chunk(e_total)

    weight, xpad = pl.pallas_call(
        _weight_xpad_kernel,
        grid=(grid,),
        in_specs=[
            pl.BlockSpec((br, c), lambda i: (i, 0)),
            pl.BlockSpec(W1.shape, lambda i: (0, 0)),
            pl.BlockSpec((1, W1.shape[1]), lambda i: (0, 0)),
            pl.BlockSpec(W2.shape, lambda i: (0, 0)),
            pl.BlockSpec((1, W2.shape[1]), lambda i: (0, 0)),
        ],
        out_specs=[
            pl.BlockSpec((br, degree), lambda i: (i, 0)),
            pl.BlockSpec((br, _F), lambda i: (i, 0)),
        ],
        out_shape=[
            jax.ShapeDtypeStruct((n, degree), jnp.float32),
            jax.ShapeDtypeStruct((npad, _F), jnp.float32),
        ],
    )(x, W1, b1.reshape(1, -1), W2, b2.reshape(1, -1))

    mesh = plsc.VectorSubcoreMesh(core_axis_name="c", subcore_axis_name="s",
                                  num_cores=_NC, num_subcores=_NS)
    sc_params = pltpu.CompilerParams(use_tc_tiling_on_sc=False,
                                     needs_layout_passes=False)
    i32 = jnp.int32
    stga = _FLUSH + ch + _CHB + 16
    stgb = _FLUSH + _CHB + _BLK + 16
    bin_fn = pl.kernel(
        functools.partial(_bin_body, e_total=e_total, ch=ch, rna=rna, rnb=rnb),
        out_type=[
            jax.ShapeDtypeStruct((_NR + _NTILES, eb), i32),
            jax.ShapeDtypeStruct((_NR + _NTILES, eb), jnp.float32),
            jax.ShapeDtypeStruct((_NR, 8), i32),
        ],
        mesh=mesh,
        scratch_types=[
            pltpu.VMEM((ch,), i32),
            pltpu.VMEM((ch,), i32),
            pltpu.VMEM((ch,), jnp.float32),
            pltpu.VMEM((stga,), i32),
            pltpu.VMEM((stga,), jnp.float32),
            pltpu.VMEM((_CHB,), i32),
            pltpu.VMEM((_CHB,), jnp.float32),
            pltpu.VMEM((_NSUB, stgb), i32),
            pltpu.VMEM((_NSUB, stgb), jnp.float32),
            pltpu.VMEM((_NSUB, 16), i32),
        ],
        compiler_params=sc_params,
    )
    bpk, bval, counts = bin_fn(edge_index[0], edge_index[1], adj_vals)

    hop_fn = pl.kernel(
        functools.partial(_hop_body, rnb=rnb, racc=racc),
        out_type=jax.ShapeDtypeStruct((npad * _F,), jnp.float32),
        mesh=mesh,
        scratch_types=[
            pltpu.VMEM((_BLK,), i32),
            pltpu.VMEM((_BLK,), jnp.float32),
            pltpu.VMEM((_BLK // _GC, _GC), i32),
            pltpu.VMEM((_GC, _F), jnp.float32),
            pltpu.VMEM((_GC, _F), jnp.float32),
            pltpu.VMEM((_GC, _F), jnp.float32),
            pltpu.VMEM((_GC, _F), jnp.float32),
            pltpu.VMEM((racc * _F,), jnp.float32),
            pltpu.VMEM((16,), i32),
            pltpu.SemaphoreType.DMA,
            pltpu.SemaphoreType.DMA,
            pltpu.SemaphoreType.DMA,
            pltpu.SemaphoreType.DMA,
        ],
        compiler_params=sc_params,
    )

    pres = []
    pre = xpad
    for _ in range(degree - 1):
        pre = hop_fn(pre, bpk, bval, counts).reshape(npad, _F)
        pres.append(pre)

    out = pl.pallas_call(
        _combine_kernel,
        grid=(grid,),
        in_specs=[pl.BlockSpec((br, c), lambda i: (i, 0))]
        + [pl.BlockSpec((br, _F), lambda i: (i, 0))] * (degree - 1)
        + [pl.BlockSpec((br, degree), lambda i: (i, 0))],
        out_specs=pl.BlockSpec((br, c), lambda i: (i, 0)),
        out_shape=jax.ShapeDtypeStruct((n, c), jnp.float32),
    )(x, *pres, weight)
    return out


# speedup vs baseline: 2.8001x; 2.7998x over previous
import functools

import jax
import jax.numpy as jnp
from jax import lax
from jax.experimental import pallas as pl
from jax.experimental.pallas import tpu as pltpu
from jax.experimental.pallas import tpu_sc as plsc

_TOPK = 16
_NC, _NS = 2, 16
_NTILES = _NC * _NS
_GC = 128
_FLUSH = 2048
_HALF = 64


def _block_rows(n):
    for br in (400, 256, 250, 200, 128, 125, 100, 64, 50, 32, 25, 16, 8, 5, 4, 2, 1):
        if n % br == 0:
            return br
    return 1


def _scan_chunk(e):
    for c in (2000, 1600, 1280, 1024, 1000, 800, 640, 512, 400, 320, 256, 160, 128, 80, 64, 32, 16):
        if e % c == 0:
            return c
    return 16


def _weight_xpad_kernel(x_ref, w1_ref, b1_ref, w2_ref, b2_ref, w_ref, xpad_ref):
    x = x_ref[...]
    br, c = x.shape
    m = jnp.max(x, axis=1, keepdims=True)
    z = jnp.sum(jnp.exp(x - m), axis=1, keepdims=True)
    cols = jax.lax.broadcasted_iota(jnp.int32, (br, c), 1)
    work = x
    tvals = []
    for _ in range(_TOPK):
        mk = jnp.max(work, axis=1, keepdims=True)
        fi = jnp.min(jnp.where(work == mk, cols, c + 1), axis=1, keepdims=True)
        tvals.append(mk)
        work = jnp.where(cols == fi, -jnp.inf, work)
    t = jnp.concatenate(tvals, axis=1)
    t = jnp.exp(t - m) / z
    h = jnp.dot(t, w1_ref[...], preferred_element_type=jnp.float32) + b1_ref[...]
    h = jnp.where(h >= 0, h, 0.1 * h)
    zz = jnp.dot(h, w2_ref[...], preferred_element_type=jnp.float32) + b2_ref[...]
    mm = jnp.max(zz, axis=1, keepdims=True)
    e = jnp.exp(zz - mm)
    w_ref[...] = e / jnp.sum(e, axis=1, keepdims=True)
    half = c // 2
    pad = jnp.zeros((br, _HALF - half), jnp.float32)
    xpad_ref[...] = jnp.stack(
        [jnp.concatenate([x[:, :half], pad], axis=1),
         jnp.concatenate([x[:, half:], pad], axis=1)], axis=0)


def _combine_kernel(x_ref, p1, p2, p3, p4, p5, w_ref, out_ref):
    x = x_ref[...]
    w = w_ref[...]
    c = x.shape[1]
    half = c // 2
    acc = w[:, 0:1] * x
    for k, p in enumerate((p1, p2, p3, p4, p5)):
        pk = jnp.concatenate([p[0][:, :half], p[1][:, :half]], axis=1)
        acc = acc + w[:, k + 1:k + 2] * pk
    m = jnp.max(acc, axis=1, keepdims=True)
    s = acc - m
    out_ref[...] = s - jnp.log(jnp.sum(jnp.exp(s), axis=1, keepdims=True))


def _bin_body(dst_hbm, src_hbm, val_hbm, bsrc, bdl, bval, counts,
              dbuf, sbuf, vbuf, sgs, sgd, sgv, cbuf, *, e_total, ch, rn):
    w = lax.axis_index("s") * _NC + lax.axis_index("c")
    lo = w * rn
    hi = lo + rn
    nch = e_total // ch

    def chunk_body(i, carry):
        cur, gcur = carry
        base = pl.multiple_of(i * ch, 8)
        pltpu.sync_copy(dst_hbm.at[pl.ds(base, ch)], dbuf)
        pltpu.sync_copy(src_hbm.at[pl.ds(base, ch)], sbuf)
        pltpu.sync_copy(val_hbm.at[pl.ds(base, ch)], vbuf)

        def vbody(j, cur):
            d = dbuf[pl.ds(j * 16, 16)]
            m = (d >= lo) & (d < hi)
            plsc.store_compressed(sgs.at[pl.ds(cur, 16)],
                                  sbuf[pl.ds(j * 16, 16)], mask=m)
            plsc.store_compressed(sgd.at[pl.ds(cur, 16)], d - lo, mask=m)
            plsc.store_compressed(sgv.at[pl.ds(cur, 16)],
                                  vbuf[pl.ds(j * 16, 16)], mask=m)
            return cur + plsc.all_reduce_population_count(m)[0]

        cur = lax.fori_loop(0, ch // 16, vbody, cur)

        def do_flush(args):
            cur, gcur = args
            gcur = pl.multiple_of(gcur, 8)
            pltpu.sync_copy(sgs.at[pl.ds(0, _FLUSH)],
                            bsrc.at[w, pl.ds(gcur, _FLUSH)])
            pltpu.sync_copy(sgd.at[pl.ds(0, _FLUSH)],
                            bdl.at[w, pl.ds(gcur, _FLUSH)])
            pltpu.sync_copy(sgv.at[pl.ds(0, _FLUSH)],
                            bval.at[w, pl.ds(gcur, _FLUSH)])

            def mv(j, _):
                sgs[pl.ds(j * 16, 16)] = sgs[pl.ds(_FLUSH + j * 16, 16)]
                sgd[pl.ds(j * 16, 16)] = sgd[pl.ds(_FLUSH + j * 16, 16)]
                sgv[pl.ds(j * 16, 16)] = sgv[pl.ds(_FLUSH + j * 16, 16)]
                return 0

            lax.fori_loop(0, ch // 16, mv, 0)
            return cur - _FLUSH, gcur + _FLUSH

        return lax.cond(cur >= _FLUSH, do_flush, lambda a: a, (cur, gcur))

    cur, gcur = lax.fori_loop(
        0, nch, chunk_body, (jnp.int32(0), jnp.int32(0)))

    zi = jnp.zeros((16,), jnp.int32)
    zf = jnp.zeros((16,), jnp.float32)
    di = jnp.full((16,), rn, jnp.int32)
    for j in range(_GC // 16):
        sgs[pl.ds(cur + j * 16, 16)] = zi
        sgd[pl.ds(cur + j * 16, 16)] = di
        sgv[pl.ds(cur + j * 16, 16)] = zf
    cur = ((cur + _GC - 1) // _GC) * _GC

    def final_flush(args):
        cur, gcur = args
        gcur = pl.multiple_of(gcur, 8)
        pltpu.sync_copy(sgs.at[pl.ds(0, _FLUSH)],
                        bsrc.at[w, pl.ds(gcur, _FLUSH)])
        pltpu.sync_copy(sgd.at[pl.ds(0, _FLUSH)],
                        bdl.at[w, pl.ds(gcur, _FLUSH)])
        pltpu.sync_copy(sgv.at[pl.ds(0, _FLUSH)],
                        bval.at[w, pl.ds(gcur, _FLUSH)])
        return args

    lax.cond(cur > 0, final_flush, lambda a: a, (cur, gcur))
    cbuf[pl.ds(0, 16)] = jnp.full((16,), gcur + cur, jnp.int32)
    pltpu.sync_copy(cbuf.at[pl.ds(0, 8)], counts.at[w])


def _hop_body(pre, bsrc, bdl, bval, counts, out,
              sbuf, dbuf, vbuf, rows, acc, cnt8, sem, *, rn, racc):
    w = lax.axis_index("s") * _NC + lax.axis_index("c")
    lo = w * rn
    pltpu.sync_copy(counts.at[w], cnt8.at[pl.ds(0, 8)])
    cnt = cnt8[pl.ds(0, 16)][0]
    nch = cnt // _GC
    zf = jnp.zeros((16,), jnp.float32)
    for h in range(2):
        def zero_body(r, _):
            for j in range(4):
                acc[r, pl.ds(j * 16, 16)] = zf
            return 0
        lax.fori_loop(0, racc, zero_body, 0)

        def chunk_body(i, _):
            base = pl.multiple_of(i * _GC, 8)
            pltpu.sync_copy(bsrc.at[w, pl.ds(base, _GC)], sbuf)
            pltpu.sync_copy(bdl.at[w, pl.ds(base, _GC)], dbuf)
            pltpu.sync_copy(bval.at[w, pl.ds(base, _GC)], vbuf)
            pltpu.async_copy(pre.at[h].at[sbuf], rows, sem).wait()

            def fma_body(k, _):
                dlv = dbuf[pl.ds(k * 16, 16)]
                vv = vbuf[pl.ds(k * 16, 16)]
                for u in range(16):
                    e = k * 16 + u
                    dl = dlv[u]
                    v = vv[u]
                    rvs = [rows[e, pl.ds(j * 16, 16)] for j in range(4)]
                    msgs = [v * rv for rv in rvs]
                    for j in range(4):
                        plsc.addupdate(acc.at[dl, pl.ds(j * 16, 16)],
                                       msgs[j])
                return 0

            lax.fori_loop(0, _GC // 16, fma_body, 0)
            return 0

        lax.fori_loop(0, nch, chunk_body, 0)
        pltpu.sync_copy(acc.at[pl.ds(0, rn)], out.at[h].at[pl.ds(lo, rn)])


def kernel(x, adj_vals, W1, b1, W2, b2, edge_index):
    n, c = x.shape
    e_total = adj_vals.shape[0]
    degree = W2.shape[1]
    br = _block_rows(n)
    grid = n // br
    rn = -(-n // _NTILES)
    npad = _NTILES * rn
    racc = ((rn + 1 + 7) // 8) * 8
    eb = e_total + 2 * _FLUSH
    ch = _scan_chunk(e_total)

    weight, xpad = pl.pallas_call(
        _weight_xpad_kernel,
        grid=(grid,),
        in_specs=[
            pl.BlockSpec((br, c), lambda i: (i, 0)),
            pl.BlockSpec(W1.shape, lambda i: (0, 0)),
            pl.BlockSpec((1, W1.shape[1]), lambda i: (0, 0)),
            pl.BlockSpec(W2.shape, lambda i: (0, 0)),
            pl.BlockSpec((1, W2.shape[1]), lambda i: (0, 0)),
        ],
        out_specs=[
            pl.BlockSpec((br, degree), lambda i: (i, 0)),
            pl.BlockSpec((2, br, _HALF), lambda i: (0, i, 0)),
        ],
        out_shape=[
            jax.ShapeDtypeStruct((n, degree), jnp.float32),
            jax.ShapeDtypeStruct((2, npad, _HALF), jnp.float32),
        ],
    )(x, W1, b1.reshape(1, -1), W2, b2.reshape(1, -1))

    mesh = plsc.VectorSubcoreMesh(core_axis_name="c", subcore_axis_name="s",
                                  num_cores=_NC, num_subcores=_NS)
    sc_params = pltpu.CompilerParams(use_tc_tiling_on_sc=False,
                                     needs_layout_passes=False)
    i32 = jnp.int32
    bin_fn = pl.kernel(
        functools.partial(_bin_body, e_total=e_total, ch=ch, rn=rn),
        out_type=[
            jax.ShapeDtypeStruct((_NTILES, eb), i32),
            jax.ShapeDtypeStruct((_NTILES, eb), i32),
            jax.ShapeDtypeStruct((_NTILES, eb), jnp.float32),
            jax.ShapeDtypeStruct((_NTILES, 8), i32),
        ],
        mesh=mesh,
        scratch_types=[
            pltpu.VMEM((ch,), i32),
            pltpu.VMEM((ch,), i32),
            pltpu.VMEM((ch,), jnp.float32),
            pltpu.VMEM((_FLUSH + ch + _GC + 16,), i32),
            pltpu.VMEM((_FLUSH + ch + _GC + 16,), i32),
            pltpu.VMEM((_FLUSH + ch + _GC + 16,), jnp.float32),
            pltpu.VMEM((16,), i32),
        ],
        compiler_params=sc_params,
    )
    bsrc, bdl, bval, counts = bin_fn(
        edge_index[0], edge_index[1], adj_vals)

    hop_fn = pl.kernel(
        functools.partial(_hop_body, rn=rn, racc=racc),
        out_type=jax.ShapeDtypeStruct((2, npad, _HALF), jnp.float32),
        mesh=mesh,
        scratch_types=[
            pltpu.VMEM((_GC,), i32),
            pltpu.VMEM((_GC,), i32),
            pltpu.VMEM((_GC,), jnp.float32),
            pltpu.VMEM((_GC, _HALF), jnp.float32),
            pltpu.VMEM((racc, _HALF), jnp.float32),
            pltpu.VMEM((16,), i32),
            pltpu.SemaphoreType.DMA,
        ],
        compiler_params=sc_params,
    )

    pres = []
    pre = xpad
    for _ in range(degree - 1):
        pre = hop_fn(pre, bsrc, bdl, bval, counts)
        pres.append(pre)

    out = pl.pallas_call(
        _combine_kernel,
        grid=(grid,),
        in_specs=[pl.BlockSpec((br, c), lambda i: (i, 0))]
        + [pl.BlockSpec((2, br, _HALF), lambda i: (0, i, 0))] * (degree - 1)
        + [pl.BlockSpec((br, degree), lambda i: (i, 0))],
        out_specs=pl.BlockSpec((br, c), lambda i: (i, 0)),
        out_shape=jax.ShapeDtypeStruct((n, c), jnp.float32),
    )(x, *pres, weight)
    return out


# speedup vs baseline: 3.5314x; 1.2612x over previous
import functools

import jax
import jax.numpy as jnp
from jax import lax
from jax.experimental import pallas as pl
from jax.experimental.pallas import tpu as pltpu
from jax.experimental.pallas import tpu_sc as plsc

_TOPK = 16
_NC, _NS = 2, 16
_NTILES = _NC * _NS
_GC = 128
_FLUSH = 2048
_HALF = 64


def _block_rows(n):
    for br in (400, 256, 250, 200, 128, 125, 100, 64, 50, 32, 25, 16, 8, 5, 4, 2, 1):
        if n % br == 0:
            return br
    return 1


def _scan_chunk(e):
    for c in (2000, 1600, 1280, 1024, 1000, 800, 640, 512, 400, 320, 256, 160, 128, 80, 64, 32, 16):
        if e % c == 0:
            return c
    return 16


def _weight_xpad_kernel(x_ref, w1_ref, b1_ref, w2_ref, b2_ref, w_ref, xpad_ref):
    x = x_ref[...]
    br, c = x.shape
    m = jnp.max(x, axis=1, keepdims=True)
    z = jnp.sum(jnp.exp(x - m), axis=1, keepdims=True)
    cols = jax.lax.broadcasted_iota(jnp.int32, (br, c), 1)
    work = x
    tvals = []
    for _ in range(_TOPK):
        mk = jnp.max(work, axis=1, keepdims=True)
        fi = jnp.min(jnp.where(work == mk, cols, c + 1), axis=1, keepdims=True)
        tvals.append(mk)
        work = jnp.where(cols == fi, -jnp.inf, work)
    t = jnp.concatenate(tvals, axis=1)
    t = jnp.exp(t - m) / z
    h = jnp.dot(t, w1_ref[...], preferred_element_type=jnp.float32) + b1_ref[...]
    h = jnp.where(h >= 0, h, 0.1 * h)
    zz = jnp.dot(h, w2_ref[...], preferred_element_type=jnp.float32) + b2_ref[...]
    mm = jnp.max(zz, axis=1, keepdims=True)
    e = jnp.exp(zz - mm)
    w_ref[...] = e / jnp.sum(e, axis=1, keepdims=True)
    half = c // 2
    pad = jnp.zeros((br, _HALF - half), jnp.float32)
    xpad_ref[...] = jnp.stack(
        [jnp.concatenate([x[:, :half], pad], axis=1),
         jnp.concatenate([x[:, half:], pad], axis=1)], axis=0)


def _combine_kernel(x_ref, p1, p2, p3, p4, p5, w_ref, out_ref):
    x = x_ref[...]
    w = w_ref[...]
    c = x.shape[1]
    half = c // 2
    acc = w[:, 0:1] * x
    for k, p in enumerate((p1, p2, p3, p4, p5)):
        pk = jnp.concatenate([p[0][:, :half], p[1][:, :half]], axis=1)
        acc = acc + w[:, k + 1:k + 2] * pk
    m = jnp.max(acc, axis=1, keepdims=True)
    s = acc - m
    out_ref[...] = s - jnp.log(jnp.sum(jnp.exp(s), axis=1, keepdims=True))


def _bin_body(dst_hbm, src_hbm, val_hbm, bsrc, bdl, bval, counts,
              dbuf, sbuf, vbuf, sgs, sgd, sgv, cbuf, *, e_total, ch, rn):
    w = lax.axis_index("s") * _NC + lax.axis_index("c")
    lo = w * rn
    hi = lo + rn
    nch = e_total // ch

    def chunk_body(i, carry):
        cur, gcur = carry
        base = pl.multiple_of(i * ch, 8)
        pltpu.sync_copy(dst_hbm.at[pl.ds(base, ch)], dbuf)
        pltpu.sync_copy(src_hbm.at[pl.ds(base, ch)], sbuf)
        pltpu.sync_copy(val_hbm.at[pl.ds(base, ch)], vbuf)

        def vbody(j, cur):
            d = dbuf[pl.ds(j * 16, 16)]
            m = (d >= lo) & (d < hi)
            plsc.store_compressed(sgs.at[pl.ds(cur, 16)],
                                  sbuf[pl.ds(j * 16, 16)], mask=m)
            plsc.store_compressed(sgd.at[pl.ds(cur, 16)], d - lo, mask=m)
            plsc.store_compressed(sgv.at[pl.ds(cur, 16)],
                                  vbuf[pl.ds(j * 16, 16)], mask=m)
            return cur + plsc.all_reduce_population_count(m)[0]

        cur = lax.fori_loop(0, ch // 16, vbody, cur)

        def do_flush(args):
            cur, gcur = args
            gcur = pl.multiple_of(gcur, 8)
            pltpu.sync_copy(sgs.at[pl.ds(0, _FLUSH)],
                            bsrc.at[w, pl.ds(gcur, _FLUSH)])
            pltpu.sync_copy(sgd.at[pl.ds(0, _FLUSH)],
                            bdl.at[w, pl.ds(gcur, _FLUSH)])
            pltpu.sync_copy(sgv.at[pl.ds(0, _FLUSH)],
                            bval.at[w, pl.ds(gcur, _FLUSH)])

            def mv(j, _):
                sgs[pl.ds(j * 16, 16)] = sgs[pl.ds(_FLUSH + j * 16, 16)]
                sgd[pl.ds(j * 16, 16)] = sgd[pl.ds(_FLUSH + j * 16, 16)]
                sgv[pl.ds(j * 16, 16)] = sgv[pl.ds(_FLUSH + j * 16, 16)]
                return 0

            lax.fori_loop(0, ch // 16, mv, 0)
            return cur - _FLUSH, gcur + _FLUSH

        return lax.cond(cur >= _FLUSH, do_flush, lambda a: a, (cur, gcur))

    cur, gcur = lax.fori_loop(
        0, nch, chunk_body, (jnp.int32(0), jnp.int32(0)))

    zi = jnp.zeros((16,), jnp.int32)
    zf = jnp.zeros((16,), jnp.float32)
    di = jnp.full((16,), rn, jnp.int32)
    for j in range(_GC // 16):
        sgs[pl.ds(cur + j * 16, 16)] = zi
        sgd[pl.ds(cur + j * 16, 16)] = di
        sgv[pl.ds(cur + j * 16, 16)] = zf
    cur = ((cur + _GC - 1) // _GC) * _GC

    def final_flush(args):
        cur, gcur = args
        gcur = pl.multiple_of(gcur, 8)
        pltpu.sync_copy(sgs.at[pl.ds(0, _FLUSH)],
                        bsrc.at[w, pl.ds(gcur, _FLUSH)])
        pltpu.sync_copy(sgd.at[pl.ds(0, _FLUSH)],
                        bdl.at[w, pl.ds(gcur, _FLUSH)])
        pltpu.sync_copy(sgv.at[pl.ds(0, _FLUSH)],
                        bval.at[w, pl.ds(gcur, _FLUSH)])
        return args

    lax.cond(cur > 0, final_flush, lambda a: a, (cur, gcur))
    cbuf[pl.ds(0, 16)] = jnp.full((16,), gcur + cur, jnp.int32)
    pltpu.sync_copy(cbuf.at[pl.ds(0, 8)], counts.at[w])


def _hop_body(pre, bsrc, bdl, bval, counts, out,
              sbuf0, sbuf1, dbuf0, dbuf1, vbuf0, vbuf1,
              rows0, rows1, acc, cnt8, sem0, sem1, *, rn, racc):
    w = lax.axis_index("s") * _NC + lax.axis_index("c")
    lo = w * rn
    pltpu.sync_copy(counts.at[w], cnt8.at[pl.ds(0, 8)])
    cnt = cnt8[pl.ds(0, 16)][0]
    nch = cnt // _GC
    zf = jnp.zeros((16,), jnp.float32)
    sbufs = (sbuf0, sbuf1)
    dbufs = (dbuf0, dbuf1)
    vbufs = (vbuf0, vbuf1)
    rowss = (rows0, rows1)
    sems = (sem0, sem1)

    def load_issue(i, b):
        base = pl.multiple_of(i * _GC, 8)
        pltpu.sync_copy(bsrc.at[w, pl.ds(base, _GC)], sbufs[b])
        pltpu.sync_copy(bdl.at[w, pl.ds(base, _GC)], dbufs[b])
        pltpu.sync_copy(bval.at[w, pl.ds(base, _GC)], vbufs[b])

    for h in range(2):
        def zero_body(r, _):
            for j in range(4):
                acc[r, pl.ds(j * 16, 16)] = zf
            return 0
        lax.fori_loop(0, racc, zero_body, 0)

        @pl.when(nch > 0)
        def _():
            load_issue(0, 0)
            pltpu.async_copy(pre.at[h].at[sbufs[0]], rowss[0], sems[0])

        def pair_body(g, _):
            for b in range(2):
                c = g * 2 + b

                @pl.when(c < nch)
                def _(c=c, b=b):
                    nb = 1 - b

                    @pl.when(c + 1 < nch)
                    def _():
                        load_issue(c + 1, nb)
                        pltpu.async_copy(pre.at[h].at[sbufs[nb]],
                                         rowss[nb], sems[nb])

                    pltpu.make_async_copy(
                        pre.at[h].at[sbufs[b]], rowss[b], sems[b]).wait()
                    rows = rowss[b]

                    def fma_body(k, _):
                        dlv = dbufs[b][pl.ds(k * 16, 16)]
                        vv = vbufs[b][pl.ds(k * 16, 16)]
                        for u in range(16):
                            e = k * 16 + u
                            dl = dlv[u]
                            v = vv[u]
                            rvs = [rows[e, pl.ds(j * 16, 16)]
                                   for j in range(4)]
                            msgs = [v * rv for rv in rvs]
                            for j in range(4):
                                plsc.addupdate(
                                    acc.at[dl, pl.ds(j * 16, 16)], msgs[j])
                        return 0

                    lax.fori_loop(0, _GC // 16, fma_body, 0)
            return 0

        lax.fori_loop(0, (nch + 1) // 2, pair_body, 0)
        pltpu.sync_copy(acc.at[pl.ds(0, rn)], out.at[h].at[pl.ds(lo, rn)])


def kernel(x, adj_vals, W1, b1, W2, b2, edge_index):
    n, c = x.shape
    e_total = adj_vals.shape[0]
    degree = W2.shape[1]
    br = _block_rows(n)
    grid = n // br
    rn = -(-n // _NTILES)
    npad = _NTILES * rn
    racc = ((rn + 1 + 7) // 8) * 8
    eb = e_total + 2 * _FLUSH
    ch = _scan_chunk(e_total)

    weight, xpad = pl.pallas_call(
        _weight_xpad_kernel,
        grid=(grid,),
        in_specs=[
            pl.BlockSpec((br, c), lambda i: (i, 0)),
            pl.BlockSpec(W1.shape, lambda i: (0, 0)),
            pl.BlockSpec((1, W1.shape[1]), lambda i: (0, 0)),
            pl.BlockSpec(W2.shape, lambda i: (0, 0)),
            pl.BlockSpec((1, W2.shape[1]), lambda i: (0, 0)),
        ],
        out_specs=[
            pl.BlockSpec((br, degree), lambda i: (i, 0)),
            pl.BlockSpec((2, br, _HALF), lambda i: (0, i, 0)),
        ],
        out_shape=[
            jax.ShapeDtypeStruct((n, degree), jnp.float32),
            jax.ShapeDtypeStruct((2, npad, _HALF), jnp.float32),
        ],
    )(x, W1, b1.reshape(1, -1), W2, b2.reshape(1, -1))

    mesh = plsc.VectorSubcoreMesh(core_axis_name="c", subcore_axis_name="s",
                                  num_cores=_NC, num_subcores=_NS)
    sc_params = pltpu.CompilerParams(use_tc_tiling_on_sc=False,
                                     needs_layout_passes=False)
    i32 = jnp.int32
    bin_fn = pl.kernel(
        functools.partial(_bin_body, e_total=e_total, ch=ch, rn=rn),
        out_type=[
            jax.ShapeDtypeStruct((_NTILES, eb), i32),
            jax.ShapeDtypeStruct((_NTILES, eb), i32),
            jax.ShapeDtypeStruct((_NTILES, eb), jnp.float32),
            jax.ShapeDtypeStruct((_NTILES, 8), i32),
        ],
        mesh=mesh,
        scratch_types=[
            pltpu.VMEM((ch,), i32),
            pltpu.VMEM((ch,), i32),
            pltpu.VMEM((ch,), jnp.float32),
            pltpu.VMEM((_FLUSH + ch + _GC + 16,), i32),
            pltpu.VMEM((_FLUSH + ch + _GC + 16,), i32),
            pltpu.VMEM((_FLUSH + ch + _GC + 16,), jnp.float32),
            pltpu.VMEM((16,), i32),
        ],
        compiler_params=sc_params,
    )
    bsrc, bdl, bval, counts = bin_fn(
        edge_index[0], edge_index[1], adj_vals)

    hop_fn = pl.kernel(
        functools.partial(_hop_body, rn=rn, racc=racc),
        out_type=jax.ShapeDtypeStruct((2, npad, _HALF), jnp.float32),
        mesh=mesh,
        scratch_types=[
            pltpu.VMEM((_GC,), i32),
            pltpu.VMEM((_GC,), i32),
            pltpu.VMEM((_GC,), i32),
            pltpu.VMEM((_GC,), i32),
            pltpu.VMEM((_GC,), jnp.float32),
            pltpu.VMEM((_GC,), jnp.float32),
            pltpu.VMEM((_GC, _HALF), jnp.float32),
            pltpu.VMEM((_GC, _HALF), jnp.float32),
            pltpu.VMEM((racc, _HALF), jnp.float32),
            pltpu.VMEM((16,), i32),
            pltpu.SemaphoreType.DMA,
            pltpu.SemaphoreType.DMA,
        ],
        compiler_params=sc_params,
    )

    pres = []
    pre = xpad
    for _ in range(degree - 1):
        pre = hop_fn(pre, bsrc, bdl, bval, counts)
        pres.append(pre)

    out = pl.pallas_call(
        _combine_kernel,
        grid=(grid,),
        in_specs=[pl.BlockSpec((br, c), lambda i: (i, 0))]
        + [pl.BlockSpec((2, br, _HALF), lambda i: (0, i, 0))] * (degree - 1)
        + [pl.BlockSpec((br, degree), lambda i: (i, 0))],
        out_specs=pl.BlockSpec((br, c), lambda i: (i, 0)),
        out_shape=jax.ShapeDtypeStruct((n, c), jnp.float32),
    )(x, *pres, weight)
    return out
